# baseline (speedup 1.0000x reference)
import functools
from math import sqrt

import numpy as np
import jax
import jax.numpy as jnp
from jax import lax
from jax.experimental import pallas as pl
from jax.experimental.pallas import tpu as pltpu
from jax.experimental.pallas import tpu_sc as plsc

_VOCAB = 1000000
_D = 64
_SEQ = 200
_B = 1024
_PAD_IDX = 0

_NC = 2
_NS = 16
_NW = _NC * _NS
_PER_W = (_B * _SEQ) // _NW
_CHUNK = _SEQ
_NCHUNK = _PER_W // _CHUNK
_LANES = 16
_SCALE = float(sqrt(_D))


def _pe_table():
    pos = np.arange(_SEQ, dtype=np.float32)[:, None]
    i = np.arange(_D, dtype=np.float32)[None, :]
    angle_rates = 1.0 / np.power(10000.0, (2.0 * np.floor(i / 2.0)) / _D)
    angles = pos * angle_rates
    pe = np.zeros((_SEQ, _D), dtype=np.float32)
    pe[:, 0::2] = np.sin(angles[:, 0::2])
    pe[:, 1::2] = np.cos(angles[:, 1::2])
    return pe


_PE_CONST = _pe_table()

def _body(table_hbm, idx_hbm, pe_hbm, out_hbm, idx_v, pe_v, rows_v, sem):
    wid = lax.axis_index("s") * _NC + lax.axis_index("c")
    base = wid * _PER_W

    pltpu.sync_copy(idx_hbm.at[pl.ds(base, _PER_W)], idx_v.at[pl.ds(0, _PER_W)])
    pltpu.sync_copy(pe_hbm, pe_v)

    def chunk_body(c, carry):
        off = c * _CHUNK
        pltpu.async_copy(
            table_hbm.at[idx_v.at[pl.ds(off, _CHUNK)]], rows_v, sem
        ).wait()

        def row_body(r, carry2):
            idx_splat = plsc.load_gather(
                idx_v, [jnp.broadcast_to(off + r, (_LANES,))]
            )
            scale = jnp.where(
                idx_splat == _PAD_IDX, jnp.float32(0.0), jnp.float32(_SCALE)
            )
            for k in range(_D // _LANES):
                sl = pl.ds(k * _LANES, _LANES)
                rows_v[r, sl] = rows_v[r, sl] * scale + pe_v[r, sl]
            return carry2

        lax.fori_loop(0, _CHUNK, row_body, 0, unroll=2)

        pltpu.sync_copy(rows_v, out_hbm.at[pl.ds(base + off, _CHUNK)])
        return carry

    lax.fori_loop(0, _NCHUNK, chunk_body, 0)


_mesh = plsc.VectorSubcoreMesh(core_axis_name="c", subcore_axis_name="s")

_kernel_call = functools.partial(
    pl.kernel,
    mesh=_mesh,
    out_type=jax.ShapeDtypeStruct((_B * _SEQ, _D), jnp.float32),
    scratch_types=[
        pltpu.VMEM((_PER_W + _LANES,), jnp.int32),
        pltpu.VMEM((_CHUNK, _D), jnp.float32),
        pltpu.VMEM((_CHUNK, _D), jnp.float32),
        pltpu.SemaphoreType.DMA,
    ],
    compiler_params=pltpu.CompilerParams(
        needs_layout_passes=False, use_tc_tiling_on_sc=False
    ),
)(_body)


@jax.jit
def kernel(x, table):
    idx = x.reshape(_B * _SEQ).astype(jnp.int32)
    out = _kernel_call(table, idx, _PE_CONST)
    return out.reshape(_B, _SEQ, _D)


# speedup vs baseline: 1.1942x; 1.1942x over previous
import functools
from math import sqrt

import numpy as np
import jax
import jax.numpy as jnp
from jax import lax
from jax.experimental import pallas as pl
from jax.experimental.pallas import tpu as pltpu
from jax.experimental.pallas import tpu_sc as plsc

_VOCAB = 1000000
_D = 64
_SEQ = 200
_B = 1024
_PAD_IDX = 0

_NC = 2
_NS = 16
_NW = _NC * _NS
_PER_W = (_B * _SEQ) // _NW
_CHUNK = _SEQ
_NCHUNK = _PER_W // _CHUNK
_LANES = 16
_GROUPS = (_CHUNK + _LANES - 1) // _LANES
_SCALE = float(sqrt(_D))
_NBUF = 4


def _pe_table():
    pos = np.arange(_SEQ, dtype=np.float32)[:, None]
    i = np.arange(_D, dtype=np.float32)[None, :]
    angle_rates = 1.0 / np.power(10000.0, (2.0 * np.floor(i / 2.0)) / _D)
    angles = pos * angle_rates
    pe = np.zeros((_SEQ, _D), dtype=np.float32)
    pe[:, 0::2] = np.sin(angles[:, 0::2])
    pe[:, 1::2] = np.cos(angles[:, 1::2])
    return pe


_PE_CONST = _pe_table()


def _body(table_hbm, idx_hbm, pe_hbm, out_hbm, idx_v, pe_v,
          b0, b1, b2, b3, g0, g1, g2, g3, s0, s1, s2, s3):
    bufs = (b0, b1, b2, b3)
    gsem = (g0, g1, g2, g3)
    ssem = (s0, s1, s2, s3)

    wid = lax.axis_index("s") * _NC + lax.axis_index("c")
    base = wid * _PER_W

    pltpu.sync_copy(idx_hbm.at[pl.ds(base, _PER_W)], idx_v.at[pl.ds(0, _PER_W)])
    slack = idx_v[pl.ds(_PER_W, _LANES)]
    idx_v[pl.ds(_PER_W, _LANES)] = slack * 0 + 1
    pltpu.sync_copy(pe_hbm, pe_v)

    def gather_start(c, b):
        return pltpu.async_copy(
            table_hbm.at[idx_v.at[pl.ds(c * _CHUNK, _CHUNK)]], bufs[b], gsem[b]
        )

    def gather_wait(b):
        pltpu.make_async_copy(
            table_hbm.at[idx_v.at[pl.ds(0, _CHUNK)]], bufs[b], gsem[b]
        ).wait()

    def scatter_start(c, b):
        return pltpu.async_copy(
            bufs[b], out_hbm.at[pl.ds(base + c * _CHUNK, _CHUNK)], ssem[b]
        )

    def scatter_wait(b):
        pltpu.make_async_copy(pe_hbm, bufs[b], ssem[b]).wait()

    def chunk_compute(c, b):
        buf = bufs[b]
        off = c * _CHUNK

        first = idx_v[pl.ds(off, _LANES)]
        acc0 = jnp.where(first == _PAD_IDX, jnp.int32(1), jnp.int32(0))

        def scan_body(g, acc):
            idxs = idx_v[pl.ds(off + g * _LANES, _LANES)]
            return acc | jnp.where(idxs == _PAD_IDX, jnp.int32(1), jnp.int32(0))

        acc = lax.fori_loop(1, _GROUPS, scan_body, acc0)
        haspad = jnp.max(acc) > 0

        @plsc.parallel_loop(0, _CHUNK, unroll=2)
        def _(r):
            for k in range(_D // _LANES):
                sl = pl.ds(k * _LANES, _LANES)
                buf[r, sl] = buf[r, sl] * _SCALE + pe_v[r, sl]

        @pl.when(haspad)
        def _():
            def fix_row(r, carry):
                idx_splat = plsc.load_gather(
                    idx_v, [jnp.broadcast_to(off + r, (_LANES,))]
                )
                pad = idx_splat == _PAD_IDX
                for k in range(_D // _LANES):
                    sl = pl.ds(k * _LANES, _LANES)
                    buf[r, sl] = jnp.where(pad, pe_v[r, sl], buf[r, sl])
                return carry

            lax.fori_loop(0, _CHUNK, fix_row, 0)

    gather_start(jnp.int32(0), 0)
    gather_start(jnp.int32(1), 1)

    def outer(g, carry):
        c0 = g * _NBUF
        for b in range(_NBUF):
            c = c0 + b
            nxt = c + 2

            @pl.when(nxt < _NCHUNK)
            def _():
                @pl.when(c >= 2)
                def _():
                    scatter_wait((b + 2) % _NBUF)

                gather_start(nxt, (b + 2) % _NBUF)

            gather_wait(b)
            chunk_compute(c, b)
            scatter_start(c, b)
        return carry

    lax.fori_loop(0, _NCHUNK // _NBUF, outer, 0)

    for b in range(_NBUF):
        scatter_wait(b)


_mesh = plsc.VectorSubcoreMesh(core_axis_name="c", subcore_axis_name="s")

_kernel_call = functools.partial(
    pl.kernel,
    mesh=_mesh,
    out_type=jax.ShapeDtypeStruct((_B * _SEQ, _D), jnp.float32),
    scratch_types=[
        pltpu.VMEM((_PER_W + _LANES,), jnp.int32),
        pltpu.VMEM((_CHUNK, _D), jnp.float32),
    ]
    + [pltpu.VMEM((_CHUNK, _D), jnp.float32) for _ in range(_NBUF)]
    + [pltpu.SemaphoreType.DMA for _ in range(2 * _NBUF)],
    compiler_params=pltpu.CompilerParams(
        needs_layout_passes=False, use_tc_tiling_on_sc=False
    ),
)(_body)


@jax.jit
def kernel(x, table):
    idx = x.reshape(_B * _SEQ).astype(jnp.int32)
    out = _kernel_call(table, idx, _PE_CONST)
    return out.reshape(_B, _SEQ, _D)
